# baseline (speedup 1.0000x reference)
import functools
import jax
import jax.numpy as jnp
from jax import lax
from jax.experimental import pallas as pl
from jax.experimental.pallas import tpu as pltpu
from jax.experimental.pallas import tpu_sc as plsc

_S = 64.0
_M = 0.4
_L2E = 1.4426950408889634


def _make_sc_gather(b):
    info = plsc.get_sparse_core_info()
    nc, ns, nl = info.num_cores, info.num_subcores, info.num_lanes
    nw = nc * ns
    per = b // nw
    mesh = plsc.VectorSubcoreMesh(core_axis_name="c", subcore_axis_name="s")

    @functools.partial(
        pl.kernel,
        mesh=mesh,
        out_type=jax.ShapeDtypeStruct((b, 128), jnp.float32),
        scratch_types=[
            pltpu.VMEM((per,), jnp.int32),
            pltpu.VMEM((per, 128), jnp.float32),
            pltpu.SemaphoreType.DMA,
        ],
    )
    def sc_gather(cos_hbm, lab_hbm, out_hbm, idx_v, rows_v, sem):
        wid = lax.axis_index("s") * nc + lax.axis_index("c")
        base = wid * per
        cs = (base // 128) * 128
        pltpu.sync_copy(lab_hbm.at[pl.ds(base, per)], idx_v)
        for k in range(per // nl):
            sl = pl.ds(k * nl, nl)
            idx_v[sl] = jnp.maximum(idx_v[sl], 0)
        pltpu.async_copy(cos_hbm.at[idx_v, pl.ds(cs, 128)], rows_v, sem).wait()
        pltpu.sync_copy(rows_v, out_hbm.at[pl.ds(base, per), :])

    return sc_gather


def _dense_kernel(cos_ref, out_ref, acc_ref, *, bcr, c_total, nblocks):
    pid = pl.program_id(0)
    b = cos_ref.shape[1]

    @pl.when(pid == 0)
    def _init():
        acc_ref[...] = jnp.zeros_like(acc_ref)

    def do_block(masked):
        acc = acc_ref[...]
        ids = jax.lax.broadcasted_iota(jnp.int32, (8, b), 0) + pid * bcr
        for s in range(bcr // 8):
            x = cos_ref[s * 8 : (s + 1) * 8, :]
            e = jnp.exp2(x * (_S * _L2E))
            if masked:
                e = jnp.where(ids < c_total, e, 0.0)
                ids = ids + 8
            acc = acc + e
        acc_ref[...] = acc

    @pl.when(pid < nblocks - 1)
    def _main():
        do_block(False)

    @pl.when(pid == nblocks - 1)
    def _last():
        do_block(True)
        out_ref[...] = jnp.sum(acc_ref[...], axis=0, keepdims=True)


def _dense_sumexp(cos_t):
    c, b = cos_t.shape
    bcr = 1024
    nblocks = pl.cdiv(c, bcr)
    return pl.pallas_call(
        functools.partial(_dense_kernel, bcr=bcr, c_total=c, nblocks=nblocks),
        grid=(nblocks,),
        in_specs=[pl.BlockSpec((bcr, b), lambda i: (i, 0))],
        out_specs=pl.BlockSpec((1, b), lambda i: (0, 0)),
        out_shape=jax.ShapeDtypeStruct((1, b), jnp.float32),
        scratch_shapes=[pltpu.VMEM((8, b), jnp.float32)],
        compiler_params=pltpu.CompilerParams(
            dimension_semantics=("arbitrary",),
        ),
    )(cos_t)


def _epi_kernel(acc_ref, rows_ref, lab_ref, out_ref):
    accrow = acc_ref[...]
    b = accrow.shape[1]
    rows = rows_ref[...]
    pick = jax.lax.broadcasted_iota(jnp.int32, (b, 128), 1) == (
        jax.lax.broadcasted_iota(jnp.int32, (b, 128), 0) % 128
    )
    xcol = jnp.sum(jnp.where(pick, rows, 0.0), axis=1)
    x = xcol.reshape(1, b)
    tl = x * (_S * _L2E)
    el = jnp.exp2(tl)
    km = 2.0 ** (-_S * _M * _L2E)
    adj = accrow - el + el * km
    valid = lab_ref[...] != -1
    nll = jnp.log(adj) - (x * _S - _S * _M)
    nll = jnp.where(valid, nll, 0.0)
    nv = jnp.maximum(jnp.sum(valid.astype(jnp.float32)), 1.0)
    out_ref[...] = (jnp.sum(nll) / nv).reshape(1, 1)


def _epilogue(accrow, rowsmat, label):
    b = label.shape[0]
    return pl.pallas_call(
        _epi_kernel,
        in_specs=[
            pl.BlockSpec((1, b), lambda: (0, 0)),
            pl.BlockSpec((b, 128), lambda: (0, 0)),
            pl.BlockSpec((1, b), lambda: (0, 0)),
        ],
        out_specs=pl.BlockSpec((1, 1), lambda: (0, 0)),
        out_shape=jax.ShapeDtypeStruct((1, 1), jnp.float32),
    )(accrow, rowsmat, label[None, :])


@jax.jit
def kernel(cosine, label):
    b, c = cosine.shape
    cos_t = cosine.T
    rowsmat = _make_sc_gather(b)(cos_t, label)
    accrow = _dense_sumexp(cos_t)
    return _epilogue(accrow, rowsmat, label)[0, 0]


# speedup vs baseline: 1.1624x; 1.1624x over previous
import functools
import jax
import jax.numpy as jnp
from jax import lax
from jax.experimental import pallas as pl
from jax.experimental.pallas import tpu as pltpu
from jax.experimental.pallas import tpu_sc as plsc

_S = 64.0
_M = 0.4
_L2E = 1.4426950408889634


def _make_sc_gather(b):
    info = plsc.get_sparse_core_info()
    nc, ns, nl = info.num_cores, info.num_subcores, info.num_lanes
    nw = nc * ns
    per = b // nw
    mesh = plsc.VectorSubcoreMesh(core_axis_name="c", subcore_axis_name="s")

    @functools.partial(
        pl.kernel,
        mesh=mesh,
        out_type=jax.ShapeDtypeStruct((b, 128), jnp.float32),
        scratch_types=[
            pltpu.VMEM((per,), jnp.int32),
            pltpu.VMEM((per, 128), jnp.float32),
            pltpu.SemaphoreType.DMA,
        ],
    )
    def sc_gather(cos_hbm, lab_hbm, out_hbm, idx_v, rows_v, sem):
        wid = lax.axis_index("s") * nc + lax.axis_index("c")
        base = wid * per
        cs = (base // 128) * 128
        pltpu.sync_copy(lab_hbm.at[pl.ds(base, per)], idx_v)
        for k in range(per // nl):
            sl = pl.ds(k * nl, nl)
            idx_v[sl] = jnp.maximum(idx_v[sl], 0)
        pltpu.async_copy(cos_hbm.at[idx_v, pl.ds(cs, 128)], rows_v, sem).wait()
        pltpu.sync_copy(rows_v, out_hbm.at[pl.ds(base, per), :])

    return sc_gather


def _dense_kernel(cos_ref, out_ref, acc_ref, *, bcr, c_total, nblocks):
    pid = pl.program_id(0)
    b = cos_ref.shape[1]

    @pl.when(pid == 0)
    def _init():
        acc_ref[...] = jnp.zeros_like(acc_ref)

    def do_block(masked):
        acc = acc_ref[...]
        ids = jax.lax.broadcasted_iota(jnp.int32, (8, b), 0) + pid * bcr
        for s in range(bcr // 8):
            x = cos_ref[s * 8 : (s + 1) * 8, :]
            e = jnp.exp2(x * (_S * _L2E))
            if masked:
                e = jnp.where(ids < c_total, e, 0.0)
                ids = ids + 8
            acc = acc + e
        acc_ref[...] = acc

    @pl.when(pid < nblocks - 1)
    def _main():
        do_block(False)

    @pl.when(pid == nblocks - 1)
    def _last():
        do_block(True)
        out_ref[...] = jnp.sum(acc_ref[...], axis=0, keepdims=True)


def _dense_sumexp(cos_t):
    c, b = cos_t.shape
    bcr = 2048
    nblocks = pl.cdiv(c, bcr)
    return pl.pallas_call(
        functools.partial(_dense_kernel, bcr=bcr, c_total=c, nblocks=nblocks),
        grid=(nblocks,),
        in_specs=[pl.BlockSpec((bcr, b), lambda i: (i, 0))],
        out_specs=pl.BlockSpec((1, b), lambda i: (0, 0)),
        out_shape=jax.ShapeDtypeStruct((1, b), jnp.float32),
        scratch_shapes=[pltpu.VMEM((8, b), jnp.float32)],
        compiler_params=pltpu.CompilerParams(
            dimension_semantics=("arbitrary",),
        ),
    )(cos_t)


def _epi_kernel(acc_ref, rows_ref, lab_ref, out_ref):
    accrow = acc_ref[...]
    b = accrow.shape[1]
    rows = rows_ref[...]
    pick = jax.lax.broadcasted_iota(jnp.int32, (b, 128), 1) == (
        jax.lax.broadcasted_iota(jnp.int32, (b, 128), 0) % 128
    )
    xcol = jnp.sum(jnp.where(pick, rows, 0.0), axis=1)
    x = xcol.reshape(1, b)
    tl = x * (_S * _L2E)
    el = jnp.exp2(tl)
    km = 2.0 ** (-_S * _M * _L2E)
    adj = accrow - el + el * km
    valid = lab_ref[...] != -1
    nll = jnp.log(adj) - (x * _S - _S * _M)
    nll = jnp.where(valid, nll, 0.0)
    nv = jnp.maximum(jnp.sum(valid.astype(jnp.float32)), 1.0)
    out_ref[...] = (jnp.sum(nll) / nv).reshape(1, 1)


def _epilogue(accrow, rowsmat, label):
    b = label.shape[0]
    return pl.pallas_call(
        _epi_kernel,
        in_specs=[
            pl.BlockSpec((1, b), lambda: (0, 0)),
            pl.BlockSpec((b, 128), lambda: (0, 0)),
            pl.BlockSpec((1, b), lambda: (0, 0)),
        ],
        out_specs=pl.BlockSpec((1, 1), lambda: (0, 0)),
        out_shape=jax.ShapeDtypeStruct((1, 1), jnp.float32),
    )(accrow, rowsmat, label[None, :])


@jax.jit
def kernel(cosine, label):
    b, c = cosine.shape
    cos_t = cosine.T
    rowsmat = _make_sc_gather(b)(cos_t, label)
    accrow = _dense_sumexp(cos_t)
    return _epilogue(accrow, rowsmat, label)[0, 0]


# speedup vs baseline: 1.1767x; 1.0123x over previous
import functools
import jax
import jax.numpy as jnp
from jax import lax
from jax.experimental import pallas as pl
from jax.experimental.pallas import tpu as pltpu
from jax.experimental.pallas import tpu_sc as plsc

_S = 64.0
_M = 0.4
_L2E = 1.4426950408889634


def _make_sc_gather(b):
    info = plsc.get_sparse_core_info()
    nc, ns, nl = info.num_cores, info.num_subcores, info.num_lanes
    nw = nc * ns
    per = b // nw
    mesh = plsc.VectorSubcoreMesh(
        core_axis_name="c", subcore_axis_name="s", num_cores=1
    )

    @functools.partial(
        pl.kernel,
        mesh=mesh,
        out_type=jax.ShapeDtypeStruct((b, 128), jnp.float32),
        scratch_types=[
            pltpu.VMEM((per,), jnp.int32),
            pltpu.VMEM((per, 128), jnp.float32),
            pltpu.SemaphoreType.DMA,
        ],
    )
    def sc_gather(cos_hbm, lab_hbm, out_hbm, idx_v, rows_v, sem):
        wid = lax.axis_index("s") * nc + lax.axis_index("c")
        base = wid * per
        cs = (base // 128) * 128
        pltpu.sync_copy(lab_hbm.at[pl.ds(base, per)], idx_v)
        for k in range(per // nl):
            sl = pl.ds(k * nl, nl)
            idx_v[sl] = jnp.maximum(idx_v[sl], 0)
        pltpu.async_copy(cos_hbm.at[idx_v, pl.ds(cs, 128)], rows_v, sem).wait()
        pltpu.sync_copy(rows_v, out_hbm.at[pl.ds(base, per), :])

    return sc_gather


def _dense_kernel(cos_ref, out_ref, acc_ref, *, bcr, c_total, nblocks):
    pid = pl.program_id(0)
    b = cos_ref.shape[1]

    @pl.when(pid == 0)
    def _init():
        acc_ref[...] = jnp.zeros_like(acc_ref)

    def do_block(masked):
        acc = acc_ref[...]
        ids = jax.lax.broadcasted_iota(jnp.int32, (8, b), 0) + pid * bcr
        for s in range(bcr // 8):
            x = cos_ref[s * 8 : (s + 1) * 8, :]
            e = jnp.exp2(x * (_S * _L2E))
            if masked:
                e = jnp.where(ids < c_total, e, 0.0)
                ids = ids + 8
            acc = acc + e
        acc_ref[...] = acc

    @pl.when(pid < nblocks - 1)
    def _main():
        do_block(False)

    @pl.when(pid == nblocks - 1)
    def _last():
        do_block(True)
        out_ref[...] = jnp.sum(acc_ref[...], axis=0, keepdims=True)


def _dense_sumexp(cos_t):
    c, b = cos_t.shape
    bcr = 2048
    nblocks = pl.cdiv(c, bcr)
    return pl.pallas_call(
        functools.partial(_dense_kernel, bcr=bcr, c_total=c, nblocks=nblocks),
        grid=(nblocks,),
        in_specs=[pl.BlockSpec((bcr, b), lambda i: (i, 0))],
        out_specs=pl.BlockSpec((1, b), lambda i: (0, 0)),
        out_shape=jax.ShapeDtypeStruct((1, b), jnp.float32),
        scratch_shapes=[pltpu.VMEM((8, b), jnp.float32)],
        compiler_params=pltpu.CompilerParams(
            dimension_semantics=("arbitrary",),
        ),
    )(cos_t)


def _epi_kernel(acc_ref, rows_ref, lab_ref, out_ref):
    accrow = acc_ref[...]
    b = accrow.shape[1]
    rows = rows_ref[...]
    pick = jax.lax.broadcasted_iota(jnp.int32, (b, 128), 1) == (
        jax.lax.broadcasted_iota(jnp.int32, (b, 128), 0) % 128
    )
    xcol = jnp.sum(jnp.where(pick, rows, 0.0), axis=1)
    x = xcol.reshape(1, b)
    tl = x * (_S * _L2E)
    el = jnp.exp2(tl)
    km = 2.0 ** (-_S * _M * _L2E)
    adj = accrow - el + el * km
    valid = lab_ref[...] != -1
    nll = jnp.log(adj) - (x * _S - _S * _M)
    nll = jnp.where(valid, nll, 0.0)
    nv = jnp.maximum(jnp.sum(valid.astype(jnp.float32)), 1.0)
    out_ref[...] = (jnp.sum(nll) / nv).reshape(1, 1)


def _epilogue(accrow, rowsmat, label):
    b = label.shape[0]
    return pl.pallas_call(
        _epi_kernel,
        in_specs=[
            pl.BlockSpec((1, b), lambda: (0, 0)),
            pl.BlockSpec((b, 128), lambda: (0, 0)),
            pl.BlockSpec((1, b), lambda: (0, 0)),
        ],
        out_specs=pl.BlockSpec((1, 1), lambda: (0, 0)),
        out_shape=jax.ShapeDtypeStruct((1, 1), jnp.float32),
    )(accrow, rowsmat, label[None, :])


@jax.jit
def kernel(cosine, label):
    b, c = cosine.shape
    cos_t = cosine.T
    rowsmat = _make_sc_gather(b)(cos_t, label)
    accrow = _dense_sumexp(cos_t)
    return _epilogue(accrow, rowsmat, label)[0, 0]
